# baseline (speedup 1.0000x reference)
import functools

import jax
import jax.numpy as jnp
from jax import lax
from jax.experimental import pallas as pl
from jax.experimental.pallas import tpu as pltpu
from jax.experimental.pallas import tpu_sc as plsc

DIM = 128
B = 4096
M = 200
NC = 2
NS = 16
NW = NC * NS
NB = B // NW
NJ = DIM // 16
CHUNK = 100
HPB = M // CHUNK
NH = NB * HPB
NBUF = 4


def _sc_body(table_hbm, uts_hbm, items_hbm, z_hbm,
             idx_v, itemidx_v, buf_v, ev_v, z_v, sems, sem_ld):
    wid = lax.axis_index("s") * NC + lax.axis_index("c")
    base = wid * NB

    pltpu.sync_copy(uts_hbm.at[wid], idx_v)
    pltpu.sync_copy(items_hbm.at[wid], itemidx_v)
    pltpu.async_copy(table_hbm.at[itemidx_v], ev_v, sem_ld)

    def fire(h, s):
        pltpu.async_copy(table_hbm.at[idx_v.at[h]], buf_v.at[s], sems[s])

    def drain(h, s):
        pltpu.make_async_copy(table_hbm.at[idx_v.at[h]], buf_v.at[s],
                              sems[s]).wait()

    for s in range(NBUF - 1):
        fire(s, s)

    pltpu.make_async_copy(table_hbm.at[itemidx_v], ev_v, sem_ld).wait()

    acc0 = tuple(jnp.zeros((16,), jnp.float32) for _ in range(NJ))

    @pl.loop(0, NH // NBUF, init_carry=acc0)
    def _(it, acc):
        h0 = it * NBUF
        for s in range(NBUF):
            h = h0 + s

            @pl.when(h + NBUF - 1 < NH)
            def _():
                fire(h + NBUF - 1, (s + NBUF - 1) % NBUF)

            drain(h, s)

            @pl.loop(0, CHUNK, init_carry=acc, unroll=4)
            def acc(r, carry):
                return tuple(carry[j] + buf_v[s, r, pl.ds(16 * j, 16)]
                             for j in range(NJ))

            if s % HPB == HPB - 1:
                i = it * (NBUF // HPB) + s // HPB
                dvec = jnp.zeros((16,), jnp.float32)
                for j in range(NJ):
                    dvec = dvec + acc[j] * ev_v[i, pl.ds(16 * j, 16)]
                z_v[i] = dvec
                acc = acc0
        return acc

    pltpu.sync_copy(z_v, z_hbm.at[pl.ds(base, NB)])


_sc_logits = functools.partial(
    pl.kernel,
    out_type=jax.ShapeDtypeStruct((B, 16), jnp.float32),
    mesh=plsc.VectorSubcoreMesh(core_axis_name="c", subcore_axis_name="s"),
    scratch_types=[
        pltpu.VMEM((NH, CHUNK), jnp.int32),
        pltpu.VMEM((NB,), jnp.int32),
        pltpu.VMEM((NBUF, CHUNK, DIM), jnp.float32),
        pltpu.VMEM((NB, DIM), jnp.float32),
        pltpu.VMEM((NB, 16), jnp.float32),
        [pltpu.SemaphoreType.DMA] * NBUF,
        pltpu.SemaphoreType.DMA,
    ],
)(_sc_body)


def _tc_body(z2_ref, y_ref, s_ref, loss_ref):
    z = jnp.sum(z2_ref[...], axis=1, keepdims=True) * (1.0 / M)
    s = jax.nn.sigmoid(z)
    s_ref[...] = s
    y = y_ref[...].astype(jnp.float32)
    eps = 1e-12
    sc = jnp.clip(s, eps, 1.0 - eps)
    loss = -jnp.mean(y * jnp.log(sc) + (1.0 - y) * jnp.log(1.0 - sc))
    loss_ref[...] = jnp.reshape(loss, (1, 1))


_tc_head = pl.pallas_call(
    _tc_body,
    out_shape=[
        jax.ShapeDtypeStruct((B, 1), jnp.float32),
        jax.ShapeDtypeStruct((1, 1), jnp.float32),
    ],
)


def kernel(entity_emb, items, labels, user_triple_set, item_triple_set):
    uts = user_triple_set[0, 0].astype(jnp.int32).reshape(NW, NH, CHUNK)
    itm = items.astype(jnp.int32).reshape(NW, NB)
    z2 = _sc_logits(entity_emb, uts, itm).reshape(B, 16)
    scores2d, loss2d = _tc_head(z2, labels.reshape(B, 1))
    return scores2d.reshape(B), loss2d[0, 0]


# speedup vs baseline: 1.0023x; 1.0023x over previous
import functools

import jax
import jax.numpy as jnp
from jax import lax
from jax.experimental import pallas as pl
from jax.experimental.pallas import tpu as pltpu
from jax.experimental.pallas import tpu_sc as plsc

DIM = 128
B = 4096
M = 200
NC = 2
NS = 16
NW = NC * NS
NB = B // NW
NJ = DIM // 16
CHUNK = 100
HPB = M // CHUNK
NH = NB * HPB
NBUF = 4


def _sc_body(table_hbm, uts_hbm, items_hbm, z_hbm,
             idx_v, itemidx_v, buf_v, ev_v, z_v, sems, sem_ld):
    wid = lax.axis_index("s") * NC + lax.axis_index("c")
    base = wid * NB

    pltpu.sync_copy(uts_hbm.at[wid], idx_v)
    pltpu.sync_copy(items_hbm.at[wid], itemidx_v)
    pltpu.async_copy(table_hbm.at[itemidx_v], ev_v, sem_ld)

    def fire(h, s):
        pltpu.async_copy(table_hbm.at[idx_v.at[h]], buf_v.at[s], sems[s])

    def drain(h, s):
        pltpu.make_async_copy(table_hbm.at[idx_v.at[h]], buf_v.at[s],
                              sems[s]).wait()

    for s in range(NBUF - 1):
        fire(s, s)

    pltpu.make_async_copy(table_hbm.at[itemidx_v], ev_v, sem_ld).wait()

    acc0 = tuple(jnp.zeros((16,), jnp.float32) for _ in range(NJ))

    @pl.loop(0, NH // NBUF, init_carry=acc0)
    def _(it, acc):
        h0 = it * NBUF
        for s in range(NBUF):
            h = h0 + s

            @pl.when(h + NBUF - 1 < NH)
            def _():
                fire(h + NBUF - 1, (s + NBUF - 1) % NBUF)

            drain(h, s)

            @pl.loop(0, CHUNK, init_carry=acc, unroll=2)
            def acc(r, carry):
                return tuple(carry[j] + buf_v[s, r, pl.ds(16 * j, 16)]
                             for j in range(NJ))

            if s % HPB == HPB - 1:
                i = it * (NBUF // HPB) + s // HPB
                dvec = jnp.zeros((16,), jnp.float32)
                for j in range(NJ):
                    dvec = dvec + acc[j] * ev_v[i, pl.ds(16 * j, 16)]
                z_v[i] = dvec
                acc = acc0
        return acc

    pltpu.sync_copy(z_v, z_hbm.at[pl.ds(base, NB)])


_sc_logits = functools.partial(
    pl.kernel,
    out_type=jax.ShapeDtypeStruct((B, 16), jnp.float32),
    mesh=plsc.VectorSubcoreMesh(core_axis_name="c", subcore_axis_name="s"),
    scratch_types=[
        pltpu.VMEM((NH, CHUNK), jnp.int32),
        pltpu.VMEM((NB,), jnp.int32),
        pltpu.VMEM((NBUF, CHUNK, DIM), jnp.float32),
        pltpu.VMEM((NB, DIM), jnp.float32),
        pltpu.VMEM((NB, 16), jnp.float32),
        [pltpu.SemaphoreType.DMA] * NBUF,
        pltpu.SemaphoreType.DMA,
    ],
)(_sc_body)


def _tc_body(z2_ref, y_ref, s_ref, loss_ref):
    z = jnp.sum(z2_ref[...], axis=1, keepdims=True) * (1.0 / M)
    s = jax.nn.sigmoid(z)
    s_ref[...] = s
    y = y_ref[...].astype(jnp.float32)
    eps = 1e-12
    sc = jnp.clip(s, eps, 1.0 - eps)
    loss = -jnp.mean(y * jnp.log(sc) + (1.0 - y) * jnp.log(1.0 - sc))
    loss_ref[...] = jnp.reshape(loss, (1, 1))


_tc_head = pl.pallas_call(
    _tc_body,
    out_shape=[
        jax.ShapeDtypeStruct((B, 1), jnp.float32),
        jax.ShapeDtypeStruct((1, 1), jnp.float32),
    ],
)


def kernel(entity_emb, items, labels, user_triple_set, item_triple_set):
    uts = user_triple_set[0, 0].astype(jnp.int32).reshape(NW, NH, CHUNK)
    itm = items.astype(jnp.int32).reshape(NW, NB)
    z2 = _sc_logits(entity_emb, uts, itm).reshape(B, 16)
    scores2d, loss2d = _tc_head(z2, labels.reshape(B, 1))
    return scores2d.reshape(B), loss2d[0, 0]
